# baseline (speedup 1.0000x reference)
import functools

import jax
import jax.numpy as jnp
from jax import lax
from jax.experimental import pallas as pl
from jax.experimental.pallas import tpu as pltpu
from jax.experimental.pallas import tpu_sc as plsc

ROWS = 128
COLS = 32768
L = 16
NC = 2
NS = 16
NW = NC * NS
ROWS_PER_W = ROWS // NW
STEPS = COLS // L


def kernel(var):
    mesh = plsc.VectorSubcoreMesh(core_axis_name="c", subcore_axis_name="s")

    @functools.partial(
        pl.kernel,
        out_type=(
            jax.ShapeDtypeStruct((ROWS * 2,), jnp.float32),
            jax.ShapeDtypeStruct((ROWS * 2,), jnp.int32),
        ),
        mesh=mesh,
        scratch_types=[
            pltpu.VMEM((COLS,), jnp.float32),
            pltpu.VMEM((COLS,), jnp.float32),
            pltpu.VMEM((L,), jnp.float32),
            pltpu.VMEM((L,), jnp.int32),
            pltpu.SemaphoreType.DMA,
            pltpu.SemaphoreType.DMA,
        ],
        compiler_params=pltpu.CompilerParams(needs_layout_passes=False),
    )
    def top2_kernel(var_hbm, vals_hbm, idxs_hbm,
                    buf0, buf1, vals_v, idxs_v, sem0, sem1):
        wid = lax.axis_index("s") * NC + lax.axis_index("c")
        row0 = wid * ROWS_PER_W
        bufs = [buf0, buf1]
        sems = [sem0, sem1]
        iota = lax.iota(jnp.int32, L)
        neg = jnp.float32(-jnp.inf)
        big = jnp.int32(2**31 - 1)

        vals_vec = jnp.zeros((L,), jnp.float32)
        idxs_vec = jnp.zeros((L,), jnp.int32)

        copies = [None, None]
        copies[0] = pltpu.async_copy(var_hbm.at[row0], bufs[0], sems[0])
        for r in range(ROWS_PER_W):
            b = r % 2
            if r + 1 < ROWS_PER_W:
                nb = (r + 1) % 2
                copies[nb] = pltpu.async_copy(
                    var_hbm.at[row0 + r + 1], bufs[nb], sems[nb])
            copies[b].wait()
            buf = bufs[b]

            def body(j, carry):
                m1, i1, m2, i2 = carry
                v = buf[pl.ds(j * L, L)]
                idx = j * L + iota
                c1 = v > m1
                c2 = v > m2
                nm2 = jnp.where(c1, m1, jnp.where(c2, v, m2))
                ni2 = jnp.where(c1, i1, jnp.where(c2, idx, i2))
                nm1 = jnp.where(c1, v, m1)
                ni1 = jnp.where(c1, idx, i1)
                return (nm1, ni1, nm2, ni2)

            init = (jnp.full((L,), neg, jnp.float32),
                    jnp.zeros((L,), jnp.int32),
                    jnp.full((L,), neg, jnp.float32),
                    jnp.zeros((L,), jnp.int32))
            m1, i1, m2, i2 = lax.fori_loop(0, STEPS, body, init)

            mx1 = jnp.max(m1)
            winm = m1 == mx1
            ix1 = jnp.min(jnp.where(winm, i1, big))
            is_win = winm & (i1 == ix1)
            cand_v = jnp.where(is_win, m2, m1)
            cand_i = jnp.where(is_win, i2, i1)
            mx2 = jnp.max(cand_v)
            ix2 = jnp.min(jnp.where(cand_v == mx2, cand_i, big))

            vals_vec = jnp.where(iota == 2 * r, mx1, vals_vec)
            vals_vec = jnp.where(iota == 2 * r + 1, mx2, vals_vec)
            idxs_vec = jnp.where(iota == 2 * r, ix1, idxs_vec)
            idxs_vec = jnp.where(iota == 2 * r + 1, ix2, idxs_vec)

        vals_v[...] = vals_vec
        idxs_v[...] = idxs_vec
        pltpu.sync_copy(vals_v.at[pl.ds(0, 2 * ROWS_PER_W)],
                        vals_hbm.at[pl.ds(row0 * 2, 2 * ROWS_PER_W)])
        pltpu.sync_copy(idxs_v.at[pl.ds(0, 2 * ROWS_PER_W)],
                        idxs_hbm.at[pl.ds(row0 * 2, 2 * ROWS_PER_W)])

    vals, idxs = top2_kernel(var)
    return vals.reshape(ROWS, 2), idxs.reshape(ROWS, 2)
